# baseline (speedup 1.0000x reference)
import functools

import jax
import jax.numpy as jnp
from jax import lax
from jax.experimental import pallas as pl

N_NEIGHBOURS = 100

Q = 4096
K = 100000
D = 128
BQ = 256
BK = 1024
K_PAD = 100352


def _dist_body(x_ref, db_ref, out_ref):
    x = x_ref[...]
    db = db_ref[...]
    cross = lax.dot_general(x, db, (((1,), (1,)), ((), ())),
                            preferred_element_type=jnp.float32)
    x_sq = jnp.sum(x * x, axis=1, keepdims=True)
    db_sq = jnp.sum(db * db, axis=1)[None, :]
    out_ref[...] = (x_sq - 2.0 * cross + db_sq) * (1.0 / D)


def kernel(X, db):
    db_p = jnp.pad(db, ((0, K_PAD - K), (0, 0)), constant_values=1e4)
    dist = pl.pallas_call(
        _dist_body,
        grid=(Q // BQ, K_PAD // BK),
        in_specs=[
            pl.BlockSpec((BQ, D), lambda i, j: (i, 0)),
            pl.BlockSpec((BK, D), lambda i, j: (j, 0)),
        ],
        out_specs=pl.BlockSpec((BQ, BK), lambda i, j: (i, j)),
        out_shape=jax.ShapeDtypeStruct((Q, K_PAD), jnp.float32),
    )(X, db_p)
    neg_vals, indices = lax.top_k(-dist, N_NEIGHBOURS)
    return -neg_vals, indices


# speedup vs baseline: 23.6256x; 23.6256x over previous
import functools

import jax
import jax.numpy as jnp
from jax import lax
from jax.experimental import pallas as pl
from jax.experimental.pallas import tpu as pltpu
from jax.experimental.pallas import tpu_sc as plsc

N_NEIGHBOURS = 100

Q = 4096
K = 100000
DIM = 128
BQ = 256
BK = 1024
NKB = 98
K_PAD = NKB * BK
G = 16
NG = K_PAD // G
GPB = BK // G

_INF = 3e38
_PADVAL = 1e4
_TAU_MARGIN = 4e-3


def _dist_body(x_ref, db_ref, d_ref, gm_ref, tau_ref,
               m1_ref, m2_ref, dbsq_r_ref, dbsq_c_ref, xsq_ref):
    i = pl.program_id(0)
    j = pl.program_id(1)
    x = x_ref[...]
    db = db_ref[...]

    @pl.when(i == 0)
    def _dbsq():
        sq = jnp.sum(db * db, axis=1, keepdims=True)
        dbsq_c_ref[j] = sq
        ones = jnp.ones((8, DIM), jnp.float32)
        row = lax.dot_general(ones, db * db, (((1,), (1,)), ((), ())),
                              preferred_element_type=jnp.float32,
                              precision=lax.Precision.HIGHEST)
        dbsq_r_ref[j] = row[0:1, :]

    @pl.when(j == 0)
    def _xsq():
        xsq_ref[...] = jnp.sum(x * x, axis=1, keepdims=True)

    x_sq = xsq_ref[...]
    dbsq_row = dbsq_r_ref[j]
    cross = lax.dot_general(x, db, (((1,), (1,)), ((), ())),
                            preferred_element_type=jnp.float32)
    dist = (x_sq - 2.0 * cross + dbsq_row) * (1.0 / DIM)
    d_ref[...] = dist

    crosst = lax.dot_general(db, x, (((1,), (1,)), ((), ())),
                             preferred_element_type=jnp.float32)
    dist_t = (dbsq_c_ref[j] - 2.0 * crosst + x_sq[:, 0][None, :]) * (1.0 / DIM)
    y = dist_t.reshape(GPB, G, BQ)
    y = jnp.minimum(y[:, 0:8, :], y[:, 8:16, :])
    gm_ref[...] = jnp.min(y, axis=1)

    @pl.when(j == 0)
    def _init():
        m1_ref[...] = jnp.full((BQ, 128), _INF, jnp.float32)
        m2_ref[...] = jnp.full((BQ, 128), _INF, jnp.float32)

    m1 = m1_ref[...]
    m2 = m2_ref[...]
    for s in range(BK // 128):
        v = dist[:, s * 128:(s + 1) * 128]
        m2 = jnp.minimum(m2, jnp.maximum(m1, v))
        m1 = jnp.minimum(m1, v)
    m1_ref[...] = m1
    m2_ref[...] = m2

    @pl.when(j == NKB - 1)
    def _finalize():
        t2 = jnp.concatenate([m1_ref[...], m2_ref[...]], axis=1)
        lo0 = jnp.min(t2, axis=1, keepdims=True)
        hi0 = jnp.max(t2, axis=1, keepdims=True)

        def body(_, carry):
            lo, hi = carry
            mid = 0.5 * (lo + hi)
            cnt = jnp.sum((t2 <= mid).astype(jnp.int32), axis=1, keepdims=True)
            ge = cnt >= N_NEIGHBOURS
            return jnp.where(ge, lo, mid), jnp.where(ge, mid, hi)

        lo, hi = lax.fori_loop(0, 30, body, (lo0, hi0))
        tau = hi + (_TAU_MARGIN + 1e-5 * jnp.abs(hi))
        tau_ref[...] = jnp.broadcast_to(tau, (BQ, 128))


def _tc_phase(X, db_p):
    return pl.pallas_call(
        _dist_body,
        grid=(Q // BQ, NKB),
        in_specs=[
            pl.BlockSpec((BQ, DIM), lambda i, j: (i, 0)),
            pl.BlockSpec((BK, DIM), lambda i, j: (j, 0)),
        ],
        out_specs=[
            pl.BlockSpec((BQ, BK), lambda i, j: (i, j)),
            pl.BlockSpec((GPB, BQ), lambda i, j: (j, i)),
            pl.BlockSpec((BQ, 128), lambda i, j: (i, 0)),
        ],
        out_shape=[
            jax.ShapeDtypeStruct((Q, K_PAD), jnp.float32),
            jax.ShapeDtypeStruct((NG, Q), jnp.float32),
            jax.ShapeDtypeStruct((Q, 128), jnp.float32),
        ],
        scratch_shapes=[
            pltpu.VMEM((BQ, 128), jnp.float32),
            pltpu.VMEM((BQ, 128), jnp.float32),
            pltpu.VMEM((NKB, 1, BK), jnp.float32),
            pltpu.VMEM((NKB, BK, 1), jnp.float32),
            pltpu.VMEM((BQ, 1), jnp.float32),
        ],
    )(X, db_p)


NW = 32
RPB = 16
NBATCH = Q // RPB // NW
CAPG = 256
GCAP = 192
CAPC = 224
PAD_ROW = NG - 1


def _sc_body(gm_hbm, tau_hbm, d4_hbm, ov_hbm, oi_hbm,
             gm_v, tau_v, cand_v, gath_v, cv_v, ci_v,
             cnt_v, outv_v, outi_v, scnt_s, stau_s, sem):
    c = lax.axis_index("c")
    s = lax.axis_index("s")
    w = s * 2 + c
    lanes = lax.iota(jnp.int32, 16)
    zero16 = jnp.zeros((16,), jnp.int32)

    def batch_body(b, _):
        q0 = (w * NBATCH + b) * RPB
        pltpu.sync_copy(gm_hbm.at[:, pl.ds(q0, RPB)], gm_v)
        pltpu.sync_copy(tau_hbm.at[pl.ds(q0, RPB)], tau_v)
        tau_vec = tau_v[...]
        qrow = (q0 + lanes) * NG
        lbase = lanes * CAPG

        def clr(t, _):
            cand_v[pl.ds(t * 16, 16)] = jnp.full((16,), PAD_ROW, jnp.int32)
            return 0
        lax.fori_loop(0, RPB * CAPG // 16, clr, 0)

        def scan_step(g, off):
            v = gm_v[g, :]
            m = jnp.logical_and(v <= tau_vec, off < CAPG)
            idxc = lbase + jnp.minimum(off, CAPG - 1)
            plsc.store_scatter(cand_v, [idxc], qrow + g, mask=m)
            return off + jnp.where(m, 1, zero16)
        offv = lax.fori_loop(0, NG, scan_step, zero16)
        cnt_v[...] = offv
        cntv = cnt_v[...]
        tauv = tau_v[...]
        for t in range(RPB):
            scnt_s[t] = jnp.minimum(cntv[t], GCAP)
            stau_s[t] = tauv[t]

        def q_body(qi, _):
            q = q0 + qi
            ng = scnt_s[qi]
            tau_qs = jnp.full((16,), stau_s[qi])
            qrow0 = q * NG

            pltpu.async_copy(
                d4_hbm.at[cand_v.at[pl.ds(qi * CAPG, 128)]],
                gath_v.at[pl.ds(0, 128)], sem).wait()

            @pl.when(ng > 128)
            def _g2():
                pltpu.async_copy(
                    d4_hbm.at[cand_v.at[pl.ds(qi * CAPG + 128, 64)]],
                    gath_v.at[pl.ds(128, 64)], sem).wait()

            def cclr(t, _):
                cv_v[pl.ds(t * 16, 16)] = jnp.full((16,), 1e9, jnp.float32)
                ci_v[pl.ds(t * 16, 16)] = zero16
                return 0
            lax.fori_loop(0, CAPC // 16, cclr, 0)

            nb = (ng + 15) // 16

            def f_blk(blk, off2):
                cnd = cand_v[pl.ds(qi * CAPG + blk * 16, 16)]
                for l in range(16):
                    v = blk * 16 + l
                    rid = cnd[l]
                    eb = (rid - qrow0) * 16
                    vals = gath_v[v, :]
                    m = jnp.logical_and(vals <= tau_qs,
                                        jnp.full((16,), v < ng))
                    off2c = jnp.minimum(off2, CAPC - 16)
                    ev = jnp.full((16,), eb, jnp.int32) + lanes
                    plsc.store_compressed(cv_v.at[pl.ds(off2c, 16)],
                                          vals, mask=m)
                    plsc.store_compressed(ci_v.at[pl.ds(off2c, 16)],
                                          ev, mask=m)
                    pc = plsc.all_reduce_population_count(m)
                    off2 = off2 + pc[0]
                return off2
            ncand = lax.fori_loop(0, nb, f_blk, 0)
            ncand = jnp.minimum(ncand, CAPC)

            nvv = (ncand + 15) // 16

            def r_outer(iv, _):
                vi = cv_v[pl.ds(iv * 16, 16)]
                ii = ci_v[pl.ds(iv * 16, 16)]

                def r_blk(jb, acc):
                    vjv = cv_v[pl.ds(jb * 16, 16)]
                    ijv = ci_v[pl.ds(jb * 16, 16)]
                    for l in range(16):
                        vjs = jnp.full((16,), vjv[l])
                        ijs = jnp.full((16,), ijv[l])
                        less = jnp.logical_or(
                            vjs < vi,
                            jnp.logical_and(vjs == vi, ijs < ii))
                        acc = acc + jnp.where(less, 1, zero16)
                    return acc
                rk = lax.fori_loop(0, nvv, r_blk, zero16)
                m = rk < 128
                plsc.store_scatter(outv_v, [rk], vi, mask=m)
                plsc.store_scatter(outi_v, [rk], ii, mask=m)
                return 0
            lax.fori_loop(0, nvv, r_outer, 0)

            pltpu.sync_copy(outv_v, ov_hbm.at[q])
            pltpu.sync_copy(outi_v, oi_hbm.at[q])
            return 0
        lax.fori_loop(0, RPB, q_body, 0)
        return 0
    lax.fori_loop(0, NBATCH, batch_body, 0)


def _sc_phase(gm, tau, d4):
    mesh = plsc.VectorSubcoreMesh(core_axis_name="c", subcore_axis_name="s")
    f = pl.kernel(
        _sc_body,
        out_type=[
            jax.ShapeDtypeStruct((Q, 128), jnp.float32),
            jax.ShapeDtypeStruct((Q, 128), jnp.int32),
        ],
        mesh=mesh,
        compiler_params=pltpu.CompilerParams(use_tc_tiling_on_sc=False,
                                             needs_layout_passes=False),
        scratch_types=[
            pltpu.VMEM((NG, RPB), jnp.float32),
            pltpu.VMEM((RPB,), jnp.float32),
            pltpu.VMEM((RPB * CAPG,), jnp.int32),
            pltpu.VMEM((GCAP, 16), jnp.float32),
            pltpu.VMEM((CAPC,), jnp.float32),
            pltpu.VMEM((CAPC,), jnp.int32),
            pltpu.VMEM((RPB,), jnp.int32),
            pltpu.VMEM((128,), jnp.float32),
            pltpu.VMEM((128,), jnp.int32),
            pltpu.SMEM((RPB,), jnp.int32),
            pltpu.SMEM((RPB,), jnp.float32),
            pltpu.SemaphoreType.DMA,
        ],
    )
    return f(gm, tau, d4)


def kernel(X, db):
    db_p = jnp.pad(db, ((0, K_PAD - K), (0, 0)), constant_values=_PADVAL)
    dist, gm, tau = _tc_phase(X, db_p)
    tau1 = tau[:, 0]
    d4 = dist.reshape(Q * K_PAD // 16, 16)
    ov, oi = _sc_phase(gm, tau1, d4)
    return ov[:, :N_NEIGHBOURS], oi[:, :N_NEIGHBOURS]


# speedup vs baseline: 23.9999x; 1.0158x over previous
import functools

import jax
import jax.numpy as jnp
from jax import lax
from jax.experimental import pallas as pl
from jax.experimental.pallas import tpu as pltpu
from jax.experimental.pallas import tpu_sc as plsc

N_NEIGHBOURS = 100

Q = 4096
K = 100000
DIM = 128
BQ = 256
BK = 1024
NKB = 98
K_PAD = NKB * BK
G = 16
NG = K_PAD // G
GPB = BK // G

_INF = 3e38
_PADVAL = 1e4
_TAU_MARGIN = 4e-3


def _dist_body(x_ref, db_ref, d_ref, gm_ref, tau_ref,
               m1_ref, m2_ref, dbsq_r_ref, dbsq_c_ref, xsq_ref,
               xs_ref, xsqr_ref):
    i = pl.program_id(0)
    j = pl.program_id(1)
    db = db_ref[...]

    @pl.when(i == 0)
    def _dbsq():
        sq = jnp.sum(db * db, axis=1, keepdims=True)
        dbsq_c_ref[j] = sq * (1.0 / DIM)
        ones = jnp.ones((8, DIM), jnp.float32)
        row = lax.dot_general(ones, db * db, (((1,), (1,)), ((), ())),
                              preferred_element_type=jnp.float32,
                              precision=lax.Precision.HIGHEST)
        dbsq_r_ref[j] = row[0:1, :] * (1.0 / DIM)

    @pl.when(j == 0)
    def _xsq():
        x = x_ref[...]
        xsq_ref[...] = jnp.sum(x * x, axis=1, keepdims=True) * (1.0 / DIM)
        xs_ref[...] = x * (2.0 / DIM)
        ones = jnp.ones((8, DIM), jnp.float32)
        rowq = lax.dot_general(ones, x * x, (((1,), (1,)), ((), ())),
                               preferred_element_type=jnp.float32,
                               precision=lax.Precision.HIGHEST)
        xsqr_ref[...] = rowq[0:1, :] * (1.0 / DIM)

    xs = xs_ref[...]
    x_sq = xsq_ref[...]
    dbsq_row = dbsq_r_ref[j]
    cross = lax.dot_general(xs, db, (((1,), (1,)), ((), ())),
                            preferred_element_type=jnp.float32)
    dist = (x_sq - cross) + dbsq_row
    d_ref[...] = dist

    crosst = lax.dot_general(db, xs, (((1,), (1,)), ((), ())),
                             preferred_element_type=jnp.float32)
    dist_t = (dbsq_c_ref[j] - crosst) + xsqr_ref[...]
    y = dist_t.reshape(GPB, G, BQ)
    y = jnp.minimum(y[:, 0:8, :], y[:, 8:16, :])
    gm_ref[...] = jnp.min(y, axis=1)

    @pl.when(j == 0)
    def _init():
        m1_ref[...] = jnp.full((BQ, 128), _INF, jnp.float32)
        m2_ref[...] = jnp.full((BQ, 128), _INF, jnp.float32)

    m1 = m1_ref[...]
    m2 = m2_ref[...]
    for s in range(BK // 128):
        v = dist[:, s * 128:(s + 1) * 128]
        m2 = jnp.minimum(m2, jnp.maximum(m1, v))
        m1 = jnp.minimum(m1, v)
    m1_ref[...] = m1
    m2_ref[...] = m2

    @pl.when(j == NKB - 1)
    def _finalize():
        t2 = jnp.concatenate([m1_ref[...], m2_ref[...]], axis=1)
        lo0 = jnp.min(t2, axis=1, keepdims=True)
        hi0 = jnp.max(t2, axis=1, keepdims=True)

        def body(_, carry):
            lo, hi = carry
            mid = 0.5 * (lo + hi)
            cnt = jnp.sum((t2 <= mid).astype(jnp.int32), axis=1, keepdims=True)
            ge = cnt >= N_NEIGHBOURS
            return jnp.where(ge, lo, mid), jnp.where(ge, mid, hi)

        lo, hi = lax.fori_loop(0, 30, body, (lo0, hi0))
        tau = hi + (_TAU_MARGIN + 1e-5 * jnp.abs(hi))
        tau_ref[...] = jnp.broadcast_to(tau, (BQ, 128))


def _tc_phase(X, db_p):
    return pl.pallas_call(
        _dist_body,
        grid=(Q // BQ, NKB),
        in_specs=[
            pl.BlockSpec((BQ, DIM), lambda i, j: (i, 0)),
            pl.BlockSpec((BK, DIM), lambda i, j: (j, 0)),
        ],
        out_specs=[
            pl.BlockSpec((BQ, BK), lambda i, j: (i, j)),
            pl.BlockSpec((GPB, BQ), lambda i, j: (j, i)),
            pl.BlockSpec((BQ, 128), lambda i, j: (i, 0)),
        ],
        out_shape=[
            jax.ShapeDtypeStruct((Q, K_PAD), jnp.float32),
            jax.ShapeDtypeStruct((NG, Q), jnp.float32),
            jax.ShapeDtypeStruct((Q, 128), jnp.float32),
        ],
        scratch_shapes=[
            pltpu.VMEM((BQ, 128), jnp.float32),
            pltpu.VMEM((BQ, 128), jnp.float32),
            pltpu.VMEM((NKB, 1, BK), jnp.float32),
            pltpu.VMEM((NKB, BK, 1), jnp.float32),
            pltpu.VMEM((BQ, 1), jnp.float32),
            pltpu.VMEM((BQ, DIM), jnp.float32),
            pltpu.VMEM((1, BQ), jnp.float32),
        ],
    )(X, db_p)


NW = 32
RPB = 16
NBATCH = Q // RPB // NW
CAPG = 256
GCAP = 192
CAPC = 224
PAD_ROW = NG - 1


def _sc_body(gm_hbm, tau_hbm, d2_hbm, ov_hbm, oi_hbm,
             gm_v, tau_v, cand_v, gath_v, cv_v, ci_v,
             cnt_v, outv_v, outi_v, scnt_s, stau_s, sem):
    c = lax.axis_index("c")
    s = lax.axis_index("s")
    w = s * 2 + c
    lanes = lax.iota(jnp.int32, 16)
    zero16 = jnp.zeros((16,), jnp.int32)
    d4_hbm = d2_hbm

    def batch_body(b, _):
        q0 = (w * NBATCH + b) * RPB
        pltpu.sync_copy(gm_hbm.at[:, pl.ds(q0, RPB)], gm_v)
        pltpu.sync_copy(tau_hbm.at[pl.ds(q0, RPB)], tau_v)
        tau_vec = tau_v[...]
        qrow = (q0 + lanes) * NG
        lbase = lanes * CAPG

        def clr(t, _):
            cand_v[pl.ds(t * 16, 16)] = jnp.full((16,), PAD_ROW, jnp.int32)
            return 0
        lax.fori_loop(0, RPB * CAPG // 16, clr, 0)

        def scan_step(g, off):
            v = gm_v[g, :]
            m = jnp.logical_and(v <= tau_vec, off < CAPG)
            idxc = lbase + jnp.minimum(off, CAPG - 1)
            plsc.store_scatter(cand_v, [idxc], qrow + g, mask=m)
            return off + jnp.where(m, 1, zero16)
        offv = lax.fori_loop(0, NG, scan_step, zero16)
        cnt_v[...] = offv
        cntv = cnt_v[...]
        tauv = tau_v[...]
        for t in range(RPB):
            scnt_s[t] = jnp.minimum(cntv[t], GCAP)
            stau_s[t] = tauv[t]

        def q_body(qi, _):
            q = q0 + qi
            ng = scnt_s[qi]
            tau_qs = jnp.full((16,), stau_s[qi])
            qrow0 = q * NG

            pltpu.async_copy(
                d4_hbm.at[cand_v.at[pl.ds(qi * CAPG, 128)]],
                gath_v.at[pl.ds(0, 128)], sem).wait()

            @pl.when(ng > 128)
            def _g2():
                pltpu.async_copy(
                    d4_hbm.at[cand_v.at[pl.ds(qi * CAPG + 128, 64)]],
                    gath_v.at[pl.ds(128, 64)], sem).wait()

            def cclr(t, _):
                cv_v[pl.ds(t * 16, 16)] = jnp.full((16,), 1e9, jnp.float32)
                ci_v[pl.ds(t * 16, 16)] = zero16
                return 0
            lax.fori_loop(0, CAPC // 16, cclr, 0)

            nb = (ng + 15) // 16

            def f_blk(blk, off2):
                cnd = cand_v[pl.ds(qi * CAPG + blk * 16, 16)]
                for l in range(16):
                    v = blk * 16 + l
                    rid = cnd[l]
                    eb = (rid - qrow0) * 16
                    vals = gath_v[v, :]
                    m = jnp.logical_and(vals <= tau_qs,
                                        jnp.full((16,), v < ng))
                    off2c = jnp.minimum(off2, CAPC - 16)
                    ev = jnp.full((16,), eb, jnp.int32) + lanes
                    plsc.store_compressed(cv_v.at[pl.ds(off2c, 16)],
                                          vals, mask=m)
                    plsc.store_compressed(ci_v.at[pl.ds(off2c, 16)],
                                          ev, mask=m)
                    pc = plsc.all_reduce_population_count(m)
                    off2 = off2 + pc[0]
                return off2
            ncand = lax.fori_loop(0, nb, f_blk, 0)
            ncand = jnp.minimum(ncand, CAPC)

            nvv = (ncand + 15) // 16

            def r_outer(iv, _):
                vi = cv_v[pl.ds(iv * 16, 16)]
                ii = ci_v[pl.ds(iv * 16, 16)]

                def r_blk(jb, acc):
                    vjv = cv_v[pl.ds(jb * 16, 16)]
                    ijv = ci_v[pl.ds(jb * 16, 16)]
                    for l in range(16):
                        vjs = jnp.full((16,), vjv[l])
                        ijs = jnp.full((16,), ijv[l])
                        less = jnp.logical_or(
                            vjs < vi,
                            jnp.logical_and(vjs == vi, ijs < ii))
                        acc = acc + jnp.where(less, 1, zero16)
                    return acc
                rk = lax.fori_loop(0, nvv, r_blk, zero16)
                m = rk < 128
                plsc.store_scatter(outv_v, [rk], vi, mask=m)
                plsc.store_scatter(outi_v, [rk], ii, mask=m)
                return 0
            lax.fori_loop(0, nvv, r_outer, 0)

            pltpu.sync_copy(outv_v, ov_hbm.at[q])
            pltpu.sync_copy(outi_v, oi_hbm.at[q])
            return 0
        lax.fori_loop(0, RPB, q_body, 0)
        return 0
    lax.fori_loop(0, NBATCH, batch_body, 0)


def _sc_phase(gm, tau, d4):
    mesh = plsc.VectorSubcoreMesh(core_axis_name="c", subcore_axis_name="s")
    f = pl.kernel(
        _sc_body,
        out_type=[
            jax.ShapeDtypeStruct((Q, 128), jnp.float32),
            jax.ShapeDtypeStruct((Q, 128), jnp.int32),
        ],
        mesh=mesh,
        compiler_params=pltpu.CompilerParams(use_tc_tiling_on_sc=False,
                                             needs_layout_passes=False),
        scratch_types=[
            pltpu.VMEM((NG, RPB), jnp.float32),
            pltpu.VMEM((RPB,), jnp.float32),
            pltpu.VMEM((RPB * CAPG,), jnp.int32),
            pltpu.VMEM((GCAP, 16), jnp.float32),
            pltpu.VMEM((CAPC,), jnp.float32),
            pltpu.VMEM((CAPC,), jnp.int32),
            pltpu.VMEM((RPB,), jnp.int32),
            pltpu.VMEM((128,), jnp.float32),
            pltpu.VMEM((128,), jnp.int32),
            pltpu.SMEM((RPB,), jnp.int32),
            pltpu.SMEM((RPB,), jnp.float32),
            pltpu.SemaphoreType.DMA,
        ],
    )
    return f(gm, tau, d4)


def kernel(X, db):
    db_p = jnp.pad(db, ((0, K_PAD - K), (0, 0)), constant_values=_PADVAL)
    dist, gm, tau = _tc_phase(X, db_p)
    tau1 = tau[:, 0]
    ov, oi = _sc_phase(gm, tau1, dist.reshape(Q * NG, G))
    return ov[:, :N_NEIGHBOURS], oi[:, :N_NEIGHBOURS]


# speedup vs baseline: 24.9798x; 1.0408x over previous
import functools

import jax
import jax.numpy as jnp
from jax import lax
from jax.experimental import pallas as pl
from jax.experimental.pallas import tpu as pltpu
from jax.experimental.pallas import tpu_sc as plsc

N_NEIGHBOURS = 100

Q = 4096
K = 100000
DIM = 128
BQ = 256
BK = 1024
NKB = 98
NQB = Q // BQ
K_PAD = NKB * BK
G = 16
NG = K_PAD // G
GPB = BK // G

_INF = 3e38
_PADVAL = 1e4
_TAU_MARGIN = 1e-3


def _dist_body(x_ref, db_ref, d_ref, gm_ref, tau_ref,
               m1_ref, m2_ref, dbsq_r_ref, dbsq_c_ref, xsq_ref,
               xsqr_ref):
    j = pl.program_id(0)
    i = pl.program_id(1)
    db = db_ref[...]

    @pl.when(i == 0)
    def _dbsq():
        sq = jnp.sum(db * db, axis=1, keepdims=True)
        dbsq_c_ref[...] = sq * (1.0 / DIM)
        ones = jnp.ones((8, DIM), jnp.float32)
        row = lax.dot_general(ones, db * db, (((1,), (1,)), ((), ())),
                              preferred_element_type=jnp.float32,
                              precision=lax.Precision.HIGHEST)
        dbsq_r_ref[...] = row[0:1, :] * (1.0 / DIM)

    @pl.when(j == 0)
    def _xsq():
        x = x_ref[...]
        xsq_ref[i] = jnp.sum(x * x, axis=1, keepdims=True) * (1.0 / DIM)
        ones = jnp.ones((8, DIM), jnp.float32)
        rowq = lax.dot_general(ones, x * x, (((1,), (1,)), ((), ())),
                               preferred_element_type=jnp.float32,
                               precision=lax.Precision.HIGHEST)
        xsqr_ref[i] = rowq[0:1, :] * (1.0 / DIM)

    xs = x_ref[...] * (2.0 / DIM)
    x_sq = xsq_ref[i]
    dbsq_row = dbsq_r_ref[...]
    cross = lax.dot_general(xs, db, (((1,), (1,)), ((), ())),
                            preferred_element_type=jnp.float32)
    dist = (x_sq - cross) + dbsq_row
    d_ref[...] = dist

    crosst = lax.dot_general(db, xs, (((1,), (1,)), ((), ())),
                             preferred_element_type=jnp.float32)
    dist_t = (dbsq_c_ref[...] - crosst) + xsqr_ref[i]
    y = dist_t.reshape(GPB, G, BQ)
    y = jnp.minimum(y[:, 0:8, :], y[:, 8:16, :])
    gm_ref[...] = jnp.min(y, axis=1)

    @pl.when(j == 0)
    def _init():
        m1_ref[i] = jnp.full((BQ, 128), _INF, jnp.float32)
        m2_ref[i] = jnp.full((BQ, 128), _INF, jnp.float32)

    m1 = m1_ref[i]
    m2 = m2_ref[i]
    for s in range(BK // 128):
        v = dist[:, s * 128:(s + 1) * 128]
        m2 = jnp.minimum(m2, jnp.maximum(m1, v))
        m1 = jnp.minimum(m1, v)
    m1_ref[i] = m1
    m2_ref[i] = m2

    @pl.when(j == NKB - 1)
    def _finalize():
        t2 = jnp.concatenate([m1_ref[i], m2_ref[i]], axis=1)
        lo0 = jnp.min(t2, axis=1, keepdims=True)
        hi0 = jnp.max(t2, axis=1, keepdims=True)

        def body(_, carry):
            lo, hi = carry
            mid = 0.5 * (lo + hi)
            cnt = jnp.sum((t2 <= mid).astype(jnp.int32), axis=1, keepdims=True)
            ge = cnt >= N_NEIGHBOURS
            return jnp.where(ge, lo, mid), jnp.where(ge, mid, hi)

        lo, hi = lax.fori_loop(0, 30, body, (lo0, hi0))
        tau = hi + (_TAU_MARGIN + 1e-5 * jnp.abs(hi))
        tau_ref[...] = jnp.broadcast_to(tau, (BQ, 128))


def _tc_phase(X, db_p):
    return pl.pallas_call(
        _dist_body,
        grid=(NKB, NQB),
        in_specs=[
            pl.BlockSpec((BQ, DIM), lambda j, i: (i, 0)),
            pl.BlockSpec((BK, DIM), lambda j, i: (j, 0)),
        ],
        out_specs=[
            pl.BlockSpec((BQ, BK), lambda j, i: (i, j)),
            pl.BlockSpec((GPB, BQ), lambda j, i: (j, i)),
            pl.BlockSpec((BQ, 128), lambda j, i: (i, 0)),
        ],
        out_shape=[
            jax.ShapeDtypeStruct((Q, K_PAD), jnp.float32),
            jax.ShapeDtypeStruct((NG, Q), jnp.float32),
            jax.ShapeDtypeStruct((Q, 128), jnp.float32),
        ],
        scratch_shapes=[
            pltpu.VMEM((NQB, BQ, 128), jnp.float32),
            pltpu.VMEM((NQB, BQ, 128), jnp.float32),
            pltpu.VMEM((1, BK), jnp.float32),
            pltpu.VMEM((BK, 1), jnp.float32),
            pltpu.VMEM((NQB, BQ, 1), jnp.float32),
            pltpu.VMEM((NQB, 1, BQ), jnp.float32),
        ],
    )(X, db_p)


NW = 32
RPB = 16
NBATCH = Q // RPB // NW
CAPG = 256
GCAP = 192
CAPC = 224
PAD_ROW = NG - 1


def _sc_body(gm_hbm, tau_hbm, d2_hbm, ov_hbm, oi_hbm,
             gm_v, tau_v, cand_v, gath_v, cv_v, ci_v,
             cnt_v, outv_v, outi_v, scnt_s, stau_s, sem):
    c = lax.axis_index("c")
    s = lax.axis_index("s")
    w = s * 2 + c
    lanes = lax.iota(jnp.int32, 16)
    zero16 = jnp.zeros((16,), jnp.int32)
    d4_hbm = d2_hbm

    def batch_body(b, _):
        q0 = (w * NBATCH + b) * RPB
        pltpu.sync_copy(gm_hbm.at[:, pl.ds(q0, RPB)], gm_v)
        pltpu.sync_copy(tau_hbm.at[pl.ds(q0, RPB)], tau_v)
        tau_vec = tau_v[...]
        qrow = (q0 + lanes) * NG
        lbase = lanes * CAPG

        def clr(t, _):
            cand_v[pl.ds(t * 16, 16)] = jnp.full((16,), PAD_ROW, jnp.int32)
            return 0
        lax.fori_loop(0, RPB * CAPG // 16, clr, 0)

        def scan_step(g, off):
            v = gm_v[g, :]
            m = jnp.logical_and(v <= tau_vec, off < CAPG)
            idxc = lbase + jnp.minimum(off, CAPG - 1)
            plsc.store_scatter(cand_v, [idxc], qrow + g, mask=m)
            return off + jnp.where(m, 1, zero16)
        offv = lax.fori_loop(0, NG, scan_step, zero16)
        cnt_v[...] = offv
        cntv = cnt_v[...]
        tauv = tau_v[...]
        for t in range(RPB):
            scnt_s[t] = jnp.minimum(cntv[t], GCAP)
            stau_s[t] = tauv[t]

        def q_body(qi, _):
            q = q0 + qi
            ng = scnt_s[qi]
            tau_qs = jnp.full((16,), stau_s[qi])
            qrow0 = q * NG

            pltpu.async_copy(
                d4_hbm.at[cand_v.at[pl.ds(qi * CAPG, 128)]],
                gath_v.at[pl.ds(0, 128)], sem).wait()

            @pl.when(ng > 128)
            def _g2():
                pltpu.async_copy(
                    d4_hbm.at[cand_v.at[pl.ds(qi * CAPG + 128, 64)]],
                    gath_v.at[pl.ds(128, 64)], sem).wait()

            def cclr(t, _):
                cv_v[pl.ds(t * 16, 16)] = jnp.full((16,), 1e9, jnp.float32)
                ci_v[pl.ds(t * 16, 16)] = zero16
                return 0
            lax.fori_loop(0, CAPC // 16, cclr, 0)

            nb = (ng + 15) // 16

            def f_blk(blk, off2):
                cnd = cand_v[pl.ds(qi * CAPG + blk * 16, 16)]
                for l in range(16):
                    v = blk * 16 + l
                    rid = cnd[l]
                    eb = (rid - qrow0) * 16
                    vals = gath_v[v, :]
                    m = jnp.logical_and(vals <= tau_qs,
                                        jnp.full((16,), v < ng))
                    off2c = jnp.minimum(off2, CAPC - 16)
                    ev = jnp.full((16,), eb, jnp.int32) + lanes
                    plsc.store_compressed(cv_v.at[pl.ds(off2c, 16)],
                                          vals, mask=m)
                    plsc.store_compressed(ci_v.at[pl.ds(off2c, 16)],
                                          ev, mask=m)
                    pc = plsc.all_reduce_population_count(m)
                    off2 = off2 + pc[0]
                return off2
            ncand = lax.fori_loop(0, nb, f_blk, 0)
            ncand = jnp.minimum(ncand, CAPC)

            nvv = (ncand + 15) // 16

            def r_outer(iv, _):
                vi = cv_v[pl.ds(iv * 16, 16)]
                ii = ci_v[pl.ds(iv * 16, 16)]

                def r_blk(jb, acc):
                    vjv = cv_v[pl.ds(jb * 16, 16)]
                    ijv = ci_v[pl.ds(jb * 16, 16)]
                    for l in range(16):
                        vjs = jnp.full((16,), vjv[l])
                        ijs = jnp.full((16,), ijv[l])
                        less = jnp.logical_or(
                            vjs < vi,
                            jnp.logical_and(vjs == vi, ijs < ii))
                        acc = acc + jnp.where(less, 1, zero16)
                    return acc
                rk = lax.fori_loop(0, nvv, r_blk, zero16)
                m = rk < 128
                plsc.store_scatter(outv_v, [rk], vi, mask=m)
                plsc.store_scatter(outi_v, [rk], ii, mask=m)
                return 0
            lax.fori_loop(0, nvv, r_outer, 0)

            pltpu.sync_copy(outv_v, ov_hbm.at[q])
            pltpu.sync_copy(outi_v, oi_hbm.at[q])
            return 0
        lax.fori_loop(0, RPB, q_body, 0)
        return 0
    lax.fori_loop(0, NBATCH, batch_body, 0)


def _sc_phase(gm, tau, d4):
    mesh = plsc.VectorSubcoreMesh(core_axis_name="c", subcore_axis_name="s")
    f = pl.kernel(
        _sc_body,
        out_type=[
            jax.ShapeDtypeStruct((Q, 128), jnp.float32),
            jax.ShapeDtypeStruct((Q, 128), jnp.int32),
        ],
        mesh=mesh,
        compiler_params=pltpu.CompilerParams(use_tc_tiling_on_sc=False,
                                             needs_layout_passes=False),
        scratch_types=[
            pltpu.VMEM((NG, RPB), jnp.float32),
            pltpu.VMEM((RPB,), jnp.float32),
            pltpu.VMEM((RPB * CAPG,), jnp.int32),
            pltpu.VMEM((GCAP, 16), jnp.float32),
            pltpu.VMEM((CAPC,), jnp.float32),
            pltpu.VMEM((CAPC,), jnp.int32),
            pltpu.VMEM((RPB,), jnp.int32),
            pltpu.VMEM((128,), jnp.float32),
            pltpu.VMEM((128,), jnp.int32),
            pltpu.SMEM((RPB,), jnp.int32),
            pltpu.SMEM((RPB,), jnp.float32),
            pltpu.SemaphoreType.DMA,
        ],
    )
    return f(gm, tau, d4)


def kernel(X, db):
    db_p = jnp.pad(db, ((0, K_PAD - K), (0, 0)), constant_values=_PADVAL)
    dist, gm, tau = _tc_phase(X, db_p)
    tau1 = tau[:, 0]
    ov, oi = _sc_phase(gm, tau1, dist.reshape(Q * NG, G))
    return ov[:, :N_NEIGHBOURS], oi[:, :N_NEIGHBOURS]


# speedup vs baseline: 33.9611x; 1.3595x over previous
import functools

import jax
import jax.numpy as jnp
from jax import lax
from jax.experimental import pallas as pl
from jax.experimental.pallas import tpu as pltpu
from jax.experimental.pallas import tpu_sc as plsc

N_NEIGHBOURS = 100

Q = 4096
K = 100000
DIM = 128
BQ = 256
BK = 1024
NKB = 98
NQB = Q // BQ
K_PAD = NKB * BK
G = 16
NG = K_PAD // G
GPB = BK // G

_INF = 3e38
_PADVAL = 1e4
_TAU_MARGIN = 1e-3


def _dist_body(x_ref, db_ref, d_ref, gm_ref, tau_ref,
               m1_ref, m2_ref, dbsq_r_ref, dbsq_c_ref, xsq_ref,
               xsqr_ref):
    j = pl.program_id(0)
    i = pl.program_id(1)
    db = db_ref[...]

    @pl.when(i == 0)
    def _dbsq():
        sq = jnp.sum(db * db, axis=1, keepdims=True)
        dbsq_c_ref[...] = sq * (1.0 / DIM)
        ones = jnp.ones((8, DIM), jnp.float32)
        row = lax.dot_general(ones, db * db, (((1,), (1,)), ((), ())),
                              preferred_element_type=jnp.float32,
                              precision=lax.Precision.HIGHEST)
        dbsq_r_ref[...] = row[0:1, :] * (1.0 / DIM)

    @pl.when(j == 0)
    def _xsq():
        x = x_ref[...]
        xsq_ref[i] = jnp.sum(x * x, axis=1, keepdims=True) * (1.0 / DIM)
        ones = jnp.ones((8, DIM), jnp.float32)
        rowq = lax.dot_general(ones, x * x, (((1,), (1,)), ((), ())),
                               preferred_element_type=jnp.float32,
                               precision=lax.Precision.HIGHEST)
        xsqr_ref[i] = rowq[0:1, :] * (1.0 / DIM)

    xs = x_ref[...] * (2.0 / DIM)
    x_sq = xsq_ref[i]
    dbsq_row = dbsq_r_ref[...]
    cross = lax.dot_general(xs, db, (((1,), (1,)), ((), ())),
                            preferred_element_type=jnp.float32)
    dist = (x_sq - cross) + dbsq_row
    d_ref[...] = dist.reshape(BQ, BK // 128, 128)

    crosst = lax.dot_general(db, xs, (((1,), (1,)), ((), ())),
                             preferred_element_type=jnp.float32)
    dist_t = (dbsq_c_ref[...] - crosst) + xsqr_ref[i]
    y = dist_t.reshape(GPB, G, BQ)
    y = jnp.minimum(y[:, 0:8, :], y[:, 8:16, :])
    gm_ref[...] = jnp.min(y, axis=1)

    @pl.when(j == 0)
    def _init():
        m1_ref[i] = jnp.full((BQ, 128), _INF, jnp.float32)
        m2_ref[i] = jnp.full((BQ, 128), _INF, jnp.float32)

    m1 = m1_ref[i]
    m2 = m2_ref[i]
    for s in range(BK // 128):
        v = dist[:, s * 128:(s + 1) * 128]
        m2 = jnp.minimum(m2, jnp.maximum(m1, v))
        m1 = jnp.minimum(m1, v)
    m1_ref[i] = m1
    m2_ref[i] = m2

    @pl.when(j == NKB - 1)
    def _finalize():
        t2 = jnp.concatenate([m1_ref[i], m2_ref[i]], axis=1)
        lo0 = jnp.min(t2, axis=1, keepdims=True)
        hi0 = jnp.max(t2, axis=1, keepdims=True)

        def body(_, carry):
            lo, hi = carry
            mid = 0.5 * (lo + hi)
            cnt = jnp.sum((t2 <= mid).astype(jnp.int32), axis=1, keepdims=True)
            ge = cnt >= N_NEIGHBOURS
            return jnp.where(ge, lo, mid), jnp.where(ge, mid, hi)

        lo, hi = lax.fori_loop(0, 30, body, (lo0, hi0))
        tau = hi + (_TAU_MARGIN + 1e-5 * jnp.abs(hi))
        tau_ref[...] = jnp.broadcast_to(tau, (BQ, 128))


def _tc_phase(X, db_p):
    return pl.pallas_call(
        _dist_body,
        grid=(NKB, NQB),
        in_specs=[
            pl.BlockSpec((BQ, DIM), lambda j, i: (i, 0)),
            pl.BlockSpec((BK, DIM), lambda j, i: (j, 0)),
        ],
        out_specs=[
            pl.BlockSpec((BQ, BK // 128, 128), lambda j, i: (i, j, 0)),
            pl.BlockSpec((GPB, BQ), lambda j, i: (j, i)),
            pl.BlockSpec((BQ, 128), lambda j, i: (i, 0)),
        ],
        out_shape=[
            jax.ShapeDtypeStruct((Q, K_PAD // 128, 128), jnp.float32),
            jax.ShapeDtypeStruct((NG, Q), jnp.float32),
            jax.ShapeDtypeStruct((Q, 128), jnp.float32),
        ],
        scratch_shapes=[
            pltpu.VMEM((NQB, BQ, 128), jnp.float32),
            pltpu.VMEM((NQB, BQ, 128), jnp.float32),
            pltpu.VMEM((1, BK), jnp.float32),
            pltpu.VMEM((BK, 1), jnp.float32),
            pltpu.VMEM((NQB, BQ, 1), jnp.float32),
            pltpu.VMEM((NQB, 1, BQ), jnp.float32),
        ],
    )(X, db_p)


NW = 32
RPB = 16
NBATCH = Q // RPB // NW
CAPG = 256
GCAP = 192
CAPC = 224
PAD_ROW = NG - 1


def _sc_body(gm_hbm, tau_hbm, d2_hbm, ov_hbm, oi_hbm,
             gm_v, tau_v, cand_v, gath_v, cv_v, ci_v,
             cnt_v, outv_v, outi_v, scnt_s, stau_s, sem):
    c = lax.axis_index("c")
    s = lax.axis_index("s")
    w = s * 2 + c
    lanes = lax.iota(jnp.int32, 16)
    zero16 = jnp.zeros((16,), jnp.int32)
    d4_hbm = d2_hbm

    def batch_body(b, _):
        q0 = (w * NBATCH + b) * RPB
        pltpu.sync_copy(gm_hbm.at[:, pl.ds(q0, RPB)], gm_v)
        pltpu.sync_copy(tau_hbm.at[pl.ds(q0, RPB)], tau_v)
        tau_vec = tau_v[...]
        qrow = (q0 + lanes) * NG
        lbase = lanes * CAPG

        def clr(t, _):
            cand_v[pl.ds(t * 16, 16)] = jnp.full((16,), PAD_ROW, jnp.int32)
            return 0
        lax.fori_loop(0, RPB * CAPG // 16, clr, 0)

        def scan_step(g, off):
            v = gm_v[g, :]
            m = jnp.logical_and(v <= tau_vec, off < CAPG)
            idxc = lbase + jnp.minimum(off, CAPG - 1)
            plsc.store_scatter(cand_v, [idxc], qrow + g, mask=m)
            return off + jnp.where(m, 1, zero16)
        offv = lax.fori_loop(0, NG, scan_step, zero16)
        cnt_v[...] = offv
        cntv = cnt_v[...]
        tauv = tau_v[...]
        for t in range(RPB):
            scnt_s[t] = jnp.minimum(cntv[t], GCAP)
            stau_s[t] = tauv[t]

        def q_body(qi, _):
            q = q0 + qi
            ng = scnt_s[qi]
            tau_qs = jnp.full((16,), stau_s[qi])
            qrow0 = q * NG

            pltpu.async_copy(
                d4_hbm.at[cand_v.at[pl.ds(qi * CAPG, 128)]],
                gath_v.at[pl.ds(0, 128)], sem).wait()

            @pl.when(ng > 128)
            def _g2():
                pltpu.async_copy(
                    d4_hbm.at[cand_v.at[pl.ds(qi * CAPG + 128, 64)]],
                    gath_v.at[pl.ds(128, 64)], sem).wait()

            def cclr(t, _):
                cv_v[pl.ds(t * 16, 16)] = jnp.full((16,), 1e9, jnp.float32)
                ci_v[pl.ds(t * 16, 16)] = zero16
                return 0
            lax.fori_loop(0, CAPC // 16, cclr, 0)

            nb = (ng + 15) // 16

            def f_blk(blk, off2):
                cnd = cand_v[pl.ds(qi * CAPG + blk * 16, 16)]
                for l in range(16):
                    v = blk * 16 + l
                    rid = cnd[l]
                    eb = (rid - qrow0) * 16
                    vals = gath_v[v, :]
                    m = jnp.logical_and(vals <= tau_qs,
                                        jnp.full((16,), v < ng))
                    off2c = jnp.minimum(off2, CAPC - 16)
                    ev = jnp.full((16,), eb, jnp.int32) + lanes
                    plsc.store_compressed(cv_v.at[pl.ds(off2c, 16)],
                                          vals, mask=m)
                    plsc.store_compressed(ci_v.at[pl.ds(off2c, 16)],
                                          ev, mask=m)
                    pc = plsc.all_reduce_population_count(m)
                    off2 = off2 + pc[0]
                return off2
            ncand = lax.fori_loop(0, nb, f_blk, 0)
            ncand = jnp.minimum(ncand, CAPC)

            nvv = (ncand + 15) // 16

            def r_outer(iv, _):
                vi = cv_v[pl.ds(iv * 16, 16)]
                ii = ci_v[pl.ds(iv * 16, 16)]

                def r_blk(jb, acc):
                    vjv = cv_v[pl.ds(jb * 16, 16)]
                    ijv = ci_v[pl.ds(jb * 16, 16)]
                    for l in range(16):
                        vjs = jnp.full((16,), vjv[l])
                        ijs = jnp.full((16,), ijv[l])
                        less = jnp.logical_or(
                            vjs < vi,
                            jnp.logical_and(vjs == vi, ijs < ii))
                        acc = acc + jnp.where(less, 1, zero16)
                    return acc
                rk = lax.fori_loop(0, nvv, r_blk, zero16)
                m = rk < 128
                plsc.store_scatter(outv_v, [rk], vi, mask=m)
                plsc.store_scatter(outi_v, [rk], ii, mask=m)
                return 0
            lax.fori_loop(0, nvv, r_outer, 0)

            pltpu.sync_copy(outv_v, ov_hbm.at[q])
            pltpu.sync_copy(outi_v, oi_hbm.at[q])
            return 0
        lax.fori_loop(0, RPB, q_body, 0)
        return 0
    lax.fori_loop(0, NBATCH, batch_body, 0)


def _sc_phase(gm, tau, d4):
    mesh = plsc.VectorSubcoreMesh(core_axis_name="c", subcore_axis_name="s")
    f = pl.kernel(
        _sc_body,
        out_type=[
            jax.ShapeDtypeStruct((Q, 128), jnp.float32),
            jax.ShapeDtypeStruct((Q, 128), jnp.int32),
        ],
        mesh=mesh,
        compiler_params=pltpu.CompilerParams(use_tc_tiling_on_sc=False,
                                             needs_layout_passes=False),
        scratch_types=[
            pltpu.VMEM((NG, RPB), jnp.float32),
            pltpu.VMEM((RPB,), jnp.float32),
            pltpu.VMEM((RPB * CAPG,), jnp.int32),
            pltpu.VMEM((GCAP, 16), jnp.float32),
            pltpu.VMEM((CAPC,), jnp.float32),
            pltpu.VMEM((CAPC,), jnp.int32),
            pltpu.VMEM((RPB,), jnp.int32),
            pltpu.VMEM((128,), jnp.float32),
            pltpu.VMEM((128,), jnp.int32),
            pltpu.SMEM((RPB,), jnp.int32),
            pltpu.SMEM((RPB,), jnp.float32),
            pltpu.SemaphoreType.DMA,
        ],
    )
    return f(gm, tau, d4)


def kernel(X, db):
    db_p = jnp.pad(db, ((0, K_PAD - K), (0, 0)), constant_values=_PADVAL)
    dist, gm, tau = _tc_phase(X, db_p)
    tau1 = tau[:, 0]
    ov, oi = _sc_phase(gm, tau1, dist.reshape(Q * NG, G))
    return ov[:, :N_NEIGHBOURS], oi[:, :N_NEIGHBOURS]
